# Initial kernel scaffold; baseline (speedup 1.0000x reference)
import jax
import jax.numpy as jnp
from jax.experimental import pallas as pl


def kernel(data, W_yr, W_mt, W_x, W_y, W_m, W_d, W_t):
    raise NotImplementedError("write your pallas kernel here")


# speedup vs baseline: 7.4942x; 7.4942x over previous
import functools

import jax
import jax.numpy as jnp
from jax import lax
from jax.experimental import pallas as pl
from jax.experimental.pallas import tpu as pltpu
from jax.experimental.pallas import tpu_sc as plsc

_B, _L, _F, _D = 4096, 200, 7, 16
_BL = _B * _L
_NW = 32
_ROWS_PER_W = _BL // _NW
_G = 128
_CHUNK = 1024
_NG = _CHUNK // _G
_NCHUNK = _ROWS_PER_W // _CHUNK


def _make_kernel():
    mesh = plsc.VectorSubcoreMesh(core_axis_name="c", subcore_axis_name="s")

    @functools.partial(
        pl.kernel,
        mesh=mesh,
        out_type=jax.ShapeDtypeStruct((_BL, _F, _D), jnp.float32),
        scratch_types=[
            pltpu.VMEM((_F, _NG, _G), jnp.int32),
            pltpu.VMEM((_F, _CHUNK, 1, _D), jnp.float32),
            pltpu.SemaphoreType.DMA,
        ],
        compiler_params=pltpu.CompilerParams(use_tc_tiling_on_sc=False),
    )
    def emb_kernel(idx_hbm, w0, w1, w2, w3, w4, w5, w6, out_hbm,
                   idx_v, rows_v, sem):
        tables = (w0, w1, w2, w3, w4, w5, w6)
        wid = lax.axis_index("s") * 2 + lax.axis_index("c")
        wbase = wid * _ROWS_PER_W

        def chunk_body(ci, carry):
            base = wbase + ci * _CHUNK
            gbase = pl.multiple_of(base // _G, 8)
            for f in range(_F):
                pltpu.sync_copy(idx_hbm.at[f, pl.ds(gbase, _NG)],
                                idx_v.at[f])
            copies = []
            for f in range(_F):
                for j in range(_NG):
                    copies.append(pltpu.async_copy(
                        tables[f].at[idx_v.at[f, j]],
                        rows_v.at[f, pl.ds(j * _G, _G), 0],
                        sem))
            for cp in copies:
                cp.wait()
            for f in range(_F):
                pltpu.sync_copy(rows_v.at[f],
                                out_hbm.at[pl.ds(base, _CHUNK), pl.ds(f, 1)])
            return carry

        lax.fori_loop(0, _NCHUNK, chunk_body, 0)

    return emb_kernel


_EMB_KERNEL = _make_kernel()


def kernel(data, W_yr, W_mt, W_x, W_y, W_m, W_d, W_t):
    idx = data.reshape(_BL, _F).T.reshape(_F, _BL // _G, _G)
    out = _EMB_KERNEL(idx, W_yr, W_mt, W_x, W_y, W_m, W_d, W_t)
    return out.reshape(_B, _L, _F * _D)


# speedup vs baseline: 7.7529x; 1.0345x over previous
import functools

import jax
import jax.numpy as jnp
from jax import lax
from jax.experimental import pallas as pl
from jax.experimental.pallas import tpu as pltpu
from jax.experimental.pallas import tpu_sc as plsc

_B, _L, _F, _D = 4096, 200, 7, 16
_BL = _B * _L
_NW = 32
_ROWS_PER_W = _BL // _NW
_G = 128
_CHUNK = 512
_NG = _CHUNK // _G
_IBLK = 8
_NCHUNK = _ROWS_PER_W // _CHUNK


def _make_kernel():
    mesh = plsc.VectorSubcoreMesh(core_axis_name="c", subcore_axis_name="s")

    @functools.partial(
        pl.kernel,
        mesh=mesh,
        out_type=jax.ShapeDtypeStruct((_BL, _F, _D), jnp.float32),
        scratch_types=[
            pltpu.VMEM((2, _F, _IBLK, _G), jnp.int32),
            pltpu.VMEM((2, _F, _CHUNK, 1, _D), jnp.float32),
            pltpu.SemaphoreType.DMA((2,)),
            pltpu.SemaphoreType.DMA((2,)),
        ],
        compiler_params=pltpu.CompilerParams(use_tc_tiling_on_sc=False),
    )
    def emb_kernel(idx_hbm, w0, w1, w2, w3, w4, w5, w6, out_hbm,
                   idx_v, rows_v, sem_g, sem_w):
        tables = (w0, w1, w2, w3, w4, w5, w6)
        wid = lax.axis_index("s") * 2 + lax.axis_index("c")
        wbase = wid * _ROWS_PER_W

        def load_idx_block(blk):
            gbase = pl.multiple_of(wbase // _G + blk * _IBLK, _IBLK)
            islot = lax.rem(blk, 2)
            for f in range(_F):
                pltpu.sync_copy(idx_hbm.at[f, pl.ds(gbase, _IBLK)],
                                idx_v.at[islot, f])

        def start_gathers(c):
            s = lax.rem(c, 2)
            islot = lax.rem(c // 2, 2)
            goff = lax.rem(c, 2) * _NG
            for f in range(_F):
                for j in range(_NG):
                    pltpu.make_async_copy(
                        tables[f].at[idx_v.at[islot, f, goff + j]],
                        rows_v.at[s, f, pl.ds(j * _G, _G), 0],
                        sem_g.at[s]).start()

        def wait_gathers(c):
            s = lax.rem(c, 2)
            for f in range(_F):
                for j in range(_NG):
                    pltpu.make_async_copy(
                        tables[f].at[idx_v.at[s, f, j]],
                        rows_v.at[s, f, pl.ds(j * _G, _G), 0],
                        sem_g.at[s]).wait()

        def start_writes(c):
            s = lax.rem(c, 2)
            base = wbase + c * _CHUNK
            for f in range(_F):
                pltpu.make_async_copy(
                    rows_v.at[s, f],
                    out_hbm.at[pl.ds(base, _CHUNK), pl.ds(f, 1)],
                    sem_w.at[s]).start()

        def wait_writes(c):
            s = lax.rem(c, 2)
            for f in range(_F):
                pltpu.make_async_copy(
                    rows_v.at[s, f],
                    out_hbm.at[pl.ds(wbase, _CHUNK), pl.ds(f, 1)],
                    sem_w.at[s]).wait()

        load_idx_block(0)
        start_gathers(0)
        start_gathers(1)
        wait_gathers(0)
        start_writes(0)

        def body(ci, carry):
            c_next = ci + 1

            @pl.when(c_next < _NCHUNK)
            def _():
                wait_writes(ci - 1)

                @pl.when(lax.rem(c_next, 2) == 0)
                def _():
                    load_idx_block(c_next // 2)

                start_gathers(c_next)

            wait_gathers(ci)
            start_writes(ci)
            return carry

        lax.fori_loop(1, _NCHUNK, body, 0)
        wait_writes(_NCHUNK - 2)
        wait_writes(_NCHUNK - 1)

    return emb_kernel


_EMB_KERNEL = _make_kernel()


def kernel(data, W_yr, W_mt, W_x, W_y, W_m, W_d, W_t):
    idx = data.reshape(_BL, _F).T.reshape(_F, _BL // _G, _G)
    out = _EMB_KERNEL(idx, W_yr, W_mt, W_x, W_y, W_m, W_d, W_t)
    return out.reshape(_B, _L, _F * _D)


# speedup vs baseline: 8.5029x; 1.0967x over previous
import functools

import jax
import jax.numpy as jnp
from jax import lax
from jax.experimental import pallas as pl
from jax.experimental.pallas import tpu as pltpu
from jax.experimental.pallas import tpu_sc as plsc

_B, _L, _F, _D = 4096, 200, 7, 16
_V = 100001
_BL_ROWS = _B * _L
_BLF = _B * _L * _F
_NW = 32
_ROWS_PER_W = _BLF // _NW
_G = 128
_CHUNK = 1024
_NG = _CHUNK // _G
_NCHUNK = _ROWS_PER_W // _CHUNK
assert _ROWS_PER_W % _CHUNK == 0


def _make_kernel(nchunk, chunk, ng):
    mesh = plsc.VectorSubcoreMesh(core_axis_name="c", subcore_axis_name="s")

    @functools.partial(
        pl.kernel,
        mesh=mesh,
        out_type=jax.ShapeDtypeStruct((_BLF, _D), jnp.float32),
        scratch_types=[
            pltpu.VMEM((2, ng, _G), jnp.int32),
            pltpu.VMEM((2, chunk, _D), jnp.float32),
            pltpu.SemaphoreType.DMA((2,)),
            pltpu.SemaphoreType.DMA((2,)),
        ],
        compiler_params=pltpu.CompilerParams(use_tc_tiling_on_sc=False),
    )
    def emb_kernel(idx_hbm, w_hbm, out_hbm, idx_v, rows_v, sem_g, sem_w):
        wid = lax.axis_index("s") * 2 + lax.axis_index("c")
        wbase = wid * _ROWS_PER_W

        def load_idx(c):
            s = lax.rem(c, 2)
            gbase = pl.multiple_of(wbase // _G + c * ng, 8)
            pltpu.sync_copy(idx_hbm.at[pl.ds(gbase, ng)], idx_v.at[s])

        def start_gathers(c):
            s = lax.rem(c, 2)
            for j in range(ng):
                pltpu.make_async_copy(
                    w_hbm.at[idx_v.at[s, j]],
                    rows_v.at[s, pl.ds(j * _G, _G)],
                    sem_g.at[s]).start()

        def wait_gathers(c):
            s = lax.rem(c, 2)
            for j in range(ng):
                pltpu.make_async_copy(
                    w_hbm.at[idx_v.at[s, j]],
                    rows_v.at[s, pl.ds(j * _G, _G)],
                    sem_g.at[s]).wait()

        def start_write(c):
            s = lax.rem(c, 2)
            base = wbase + c * chunk
            pltpu.make_async_copy(
                rows_v.at[s], out_hbm.at[pl.ds(base, chunk)],
                sem_w.at[s]).start()

        def wait_write(c):
            s = lax.rem(c, 2)
            pltpu.make_async_copy(
                rows_v.at[s], out_hbm.at[pl.ds(wbase, chunk)],
                sem_w.at[s]).wait()

        load_idx(0)
        start_gathers(0)
        load_idx(1)
        start_gathers(1)
        wait_gathers(0)
        start_write(0)

        def body(ci, carry):
            c_next = ci + 1

            @pl.when(c_next < nchunk)
            def _():
                wait_write(ci - 1)
                load_idx(c_next)
                start_gathers(c_next)

            wait_gathers(ci)
            start_write(ci)
            return carry

        lax.fori_loop(1, nchunk, body, 0)
        wait_write(nchunk - 2)
        wait_write(nchunk - 1)

    return emb_kernel


_EMB_KERNEL = _make_kernel(_ROWS_PER_W // _CHUNK, _CHUNK, _NG)


def kernel(data, W_yr, W_mt, W_x, W_y, W_m, W_d, W_t):
    offs = (jnp.arange(_F, dtype=jnp.int32) * _V)
    idx = (data.reshape(_BL_ROWS, _F) + offs).reshape(_BLF // _G, _G)
    w_all = jnp.concatenate((W_yr, W_mt, W_x, W_y, W_m, W_d, W_t), axis=0)
    out = _EMB_KERNEL(idx, w_all)
    return out.reshape(_B, _L, _F * _D)


# speedup vs baseline: 9.6116x; 1.1304x over previous
import functools

import jax
import jax.numpy as jnp
from jax import lax
from jax.experimental import pallas as pl
from jax.experimental.pallas import tpu as pltpu
from jax.experimental.pallas import tpu_sc as plsc

_B, _L, _F, _D = 4096, 200, 7, 16
_NW = 32
_BW = _B // _NW
_GL = 8
_NGRP = _L // _GL


def _make_kernel():
    mesh = plsc.VectorSubcoreMesh(core_axis_name="c", subcore_axis_name="s")

    @functools.partial(
        pl.kernel,
        mesh=mesh,
        out_type=jax.ShapeDtypeStruct((_L, _F * _D, _B), jnp.float32),
        scratch_types=[
            pltpu.VMEM((_L, _BW), jnp.int32),
            pltpu.VMEM((2, _GL * _BW, _D), jnp.float32),
            pltpu.VMEM((2, _GL, _D, _BW), jnp.float32),
            pltpu.SemaphoreType.DMA((2,)),
            pltpu.SemaphoreType.DMA((2,)),
        ],
        compiler_params=pltpu.CompilerParams(use_tc_tiling_on_sc=False,
                                             needs_layout_passes=False),
    )
    def emb_kernel(idx_hbm, w0, w1, w2, w3, w4, w5, w6, out_hbm,
                   idx_v, rows_v, trans_v, sem_g, sem_w):
        tables = (w0, w1, w2, w3, w4, w5, w6)
        wid = lax.axis_index("s") * 2 + lax.axis_index("c")
        b0 = wid * _BW
        lanes = lax.iota(jnp.int32, 16)

        for f in range(_F):
            table = tables[f]
            pltpu.sync_copy(idx_hbm.at[f, :, pl.ds(b0, _BW)], idx_v)

            def start_gathers(t):
                s = lax.rem(t, 2)

                def g_body(j, carry):
                    pltpu.make_async_copy(
                        table.at[idx_v.at[t * _GL + j]],
                        rows_v.at[s, pl.ds(j * _BW, _BW)],
                        sem_g.at[s]).start()
                    return carry

                lax.fori_loop(0, _GL, g_body, 0)

            def wait_gathers(t):
                s = lax.rem(t, 2)

                def w_body(j, carry):
                    pltpu.make_async_copy(
                        table.at[idx_v.at[j]],
                        rows_v.at[s, pl.ds(j * _BW, _BW)],
                        sem_g.at[s]).wait()
                    return carry

                lax.fori_loop(0, _GL, w_body, 0)

            def transpose_group(t):
                s = lax.rem(t, 2)
                rows2d = rows_v.at[s]

                def tr_body(i, carry):
                    j = i // (_BW // 16)
                    q = lax.rem(i, _BW // 16)
                    row_idx = lanes + (j * _BW + q * 16)
                    for d in range(_D):
                        col_idx = jnp.full((16,), d, jnp.int32)
                        vec = plsc.load_gather(rows2d, [row_idx, col_idx])
                        trans_v[s, j, d, pl.ds(q * 16, 16)] = vec
                    return carry

                lax.fori_loop(0, _GL * (_BW // 16), tr_body, 0)

            def start_write(t):
                s = lax.rem(t, 2)
                pltpu.make_async_copy(
                    trans_v.at[s],
                    out_hbm.at[pl.ds(t * _GL, _GL),
                               pl.ds(f * _D, _D),
                               pl.ds(b0, _BW)],
                    sem_w.at[s]).start()

            def wait_write(t):
                s = lax.rem(t, 2)
                pltpu.make_async_copy(
                    trans_v.at[s],
                    out_hbm.at[pl.ds(0, _GL), pl.ds(f * _D, _D),
                               pl.ds(b0, _BW)],
                    sem_w.at[s]).wait()

            start_gathers(0)

            def body(t, carry):
                @pl.when(t + 1 < _NGRP)
                def _():
                    start_gathers(t + 1)
                wait_gathers(t)

                @pl.when(t >= 2)
                def _():
                    wait_write(t - 2)
                transpose_group(t)
                start_write(t)
                return carry

            lax.fori_loop(0, _NGRP, body, 0)
            wait_write(_NGRP - 2)
            wait_write(_NGRP - 1)

    return emb_kernel


_EMB_KERNEL = _make_kernel()


def kernel(data, W_yr, W_mt, W_x, W_y, W_m, W_d, W_t):
    data_t = jnp.transpose(data, (2, 1, 0))
    out = _EMB_KERNEL(data_t, W_yr, W_mt, W_x, W_y, W_m, W_d, W_t)
    return jnp.transpose(out, (2, 0, 1))
